# baseline (speedup 1.0000x reference)
import functools

import jax
import jax.numpy as jnp
from jax import lax
from jax.experimental import pallas as pl
from jax.experimental.pallas import tpu as pltpu
from jax.experimental.pallas import tpu_sc as plsc

N = 10000
NP = 10240
E = 320000
D = 128
HALF = 64
DEGREE = 16
ALPHA = 0.6

NC = 2
NS = 16
NR = NP // NS
RQ = 5
RB = NR // RQ
CHUNK = 128
BCH = 8
NBLK = 20
NCHUNK = NBLK * BCH
E_TILE = NCHUNK * CHUNK
E_PAD = NS * E_TILE
NRING = 4
LOOK = 2

_GDN = lax.GatherDimensionNumbers(
    offset_dims=(), collapsed_slice_dims=(0,), start_index_map=(0,))


def _lane_bcast(vec, l):
    idx = jnp.full((16, 1), l, jnp.int32)
    return lax.gather(vec, idx, _GDN, (1,),
                      mode=lax.GatherScatterMode.PROMISE_IN_BOUNDS)


def _spmm_body(xs, colh, rowh, ewh, out,
               S1, S2, col_s, row_s, ew_s,
               g0, g1, g2, g3, zbuf,
               sg0, sg1, sg2, sg3, ss0, ss1, ss2, ss3, se0, se1):
    c = lax.axis_index("c")
    s = lax.axis_index("s")
    base = s * NR
    coff = c * NP
    gbufs = (g0, g1, g2, g3)
    sgs = (sg0, sg1, sg2, sg3)
    sss = (ss0, ss1, ss2, ss3)
    ses = (se0, se1)

    def zb(i, carry):
        for q2 in range(4):
            zbuf[i, pl.ds(q2 * 16, 16)] = jnp.zeros((16,), jnp.float32)
        return carry

    lax.fori_loop(0, 64, zb, 0)

    for q in range(RQ):
        pltpu.sync_copy(xs.at[pl.ds(coff + base + q * RB, RB)], g0)
        pltpu.sync_copy(g0, S1.at[pl.ds(base + q * RB, RB)])
        pltpu.sync_copy(g0, out.at[c, pl.ds(base + q * RB, RB)])
    for q in range(2 * RQ):
        pltpu.sync_copy(zbuf, S2.at[pl.ds(base + q * 64, 64)])
    plsc.subcore_barrier()

    def issue_stage(bi, half):
        pltpu.async_copy(colh.at[s, bi], col_s.at[half], ses[0])
        pltpu.async_copy(rowh.at[s, bi], row_s.at[half], ses[0])
        pltpu.async_copy(ewh.at[s, bi], ew_s.at[half], ses[1])

    def wait_stage():
        pltpu.make_async_copy(colh.at[0, 0], col_s.at[0], ses[0]).wait()
        pltpu.make_async_copy(rowh.at[0, 0], row_s.at[0], ses[0]).wait()
        pltpu.make_async_copy(ewh.at[0, 0], ew_s.at[0], ses[1]).wait()

    def wait_sg(b):
        pltpu.make_async_copy(xs.at[pl.ds(0, CHUNK)], gbufs[b], sgs[b]).wait()

    def wait_ss(b):
        pltpu.make_async_copy(xs.at[pl.ds(0, CHUNK)], gbufs[b], sss[b]).wait()

    def one_round(src, dst, asc):
        def process(gkp, pb):
            hp = lax.rem(gkp >> 3, 3)
            kp = gkp & 7
            wait_sg(pb)
            g = gbufs[pb]

            def h16(h, c3):
                wv = ew_s[hp, kp, pl.ds(h * 16, 16)]
                for dl in range(16):
                    w = _lane_bcast(wv, dl)
                    for q2 in range(4):
                        slx = pl.ds(q2 * 16, 16)
                        g[h * 16 + dl, slx] = g[h * 16 + dl, slx] * w
                return c3

            lax.fori_loop(0, CHUNK // 16, h16, 0)
            pltpu.async_copy(g, dst.at[row_s.at[hp, kp]], sss[pb], add=True)

        issue_stage(0, 0)

        def block_body(bi, carry):
            half = lax.rem(bi, 3)
            wait_stage()

            @pl.when(bi + 1 < NBLK)
            def _():
                issue_stage(bi + 1, lax.rem(bi + 1, 3))

            def kk_body(kk, c2):
                for b in range(NRING):
                    gk = bi * BCH + kk * NRING + b

                    @pl.when(gk >= NRING)
                    def _():
                        wait_ss(b)

                    pltpu.async_copy(
                        src.at[col_s.at[half, kk * NRING + b]],
                        gbufs[b], sgs[b])

                    @pl.when(gk >= LOOK)
                    def _():
                        process(gk - LOOK, (b + NRING - LOOK) % NRING)
                return c2

            lax.fori_loop(0, BCH // NRING, kk_body, 0)
            return carry

        lax.fori_loop(0, NBLK, block_body, 0)
        for t in range(LOOK):
            gkp = NCHUNK - LOOK + t
            process(gkp, gkp % NRING)
        for b in range(NRING):
            wait_ss(b)
        plsc.subcore_barrier()

        def rd(q):
            m = q % 2
            pltpu.async_copy(dst.at[pl.ds(base + q * RB, RB)],
                             gbufs[2 * m], sgs[2 * m])
            pltpu.async_copy(out.at[c, pl.ds(base + q * RB, RB)],
                             gbufs[2 * m + 1], sgs[2 * m + 1])

        rd(0)
        rd(1)
        for q in range(RQ):
            m = q % 2
            wait_sg(2 * m)
            wait_sg(2 * m + 1)
            ga = gbufs[2 * m]
            ge = gbufs[2 * m + 1]

            def eb(i, c2):
                for q2 in range(4):
                    slx = pl.ds(q2 * 16, 16)
                    ge[i, slx] = ge[i, slx] + ga[i, slx] * asc
                return c2

            lax.fori_loop(0, RB, eb, 0)
            pltpu.async_copy(ge, out.at[c, pl.ds(base + q * RB, RB)],
                             sss[2 * m + 1])
            pltpu.async_copy(zbuf, src.at[pl.ds(base + q * RB, 64)], ses[0])
            pltpu.async_copy(zbuf, src.at[pl.ds(base + q * RB + 64, 64)],
                             ses[0])
            if q + 2 <= RQ - 1:
                wait_ss(2 * m + 1)
                rd(q + 2)
        wait_ss(1)
        wait_ss(3)
        for _ in range(2 * RQ):
            pltpu.make_async_copy(zbuf, src.at[pl.ds(base, 64)],
                                  ses[0]).wait()
        plsc.subcore_barrier()

    a = jnp.float32(ALPHA)

    def two_rounds(r2, asc):
        one_round(S1, S2, asc)
        one_round(S2, S1, asc * a)
        return asc * jnp.float32(ALPHA * ALPHA)

    lax.fori_loop(0, DEGREE // 2, two_rounds, a)


_spmm_call = functools.partial(
    pl.kernel,
    out_type=jax.ShapeDtypeStruct((NC, NP, HALF), jnp.float32),
    mesh=plsc.VectorSubcoreMesh(core_axis_name="c", subcore_axis_name="s"),
    compiler_params=pltpu.CompilerParams(use_tc_tiling_on_sc=False),
    scratch_types=(
        [
            pltpu.VMEM_SHARED((NP, HALF), jnp.float32),
            pltpu.VMEM_SHARED((NP, HALF), jnp.float32),
            pltpu.VMEM((3, BCH, CHUNK), jnp.int32),
            pltpu.VMEM((3, BCH, CHUNK), jnp.int32),
            pltpu.VMEM((3, BCH, CHUNK), jnp.float32),
        ]
        + [pltpu.VMEM((CHUNK, HALF), jnp.float32)] * 4
        + [pltpu.VMEM((64, HALF), jnp.float32)]
        + [pltpu.SemaphoreType.DMA] * 10
    ),
)(_spmm_body)


def _linear_body(emb_ref, w_ref, b_ref, o_ref):
    o_ref[...] = (
        jnp.dot(emb_ref[...] * (1.0 / DEGREE), w_ref[...],
                preferred_element_type=jnp.float32)
        + b_ref[...]
    )


def _linear(emb, wt, b2):
    return pl.pallas_call(
        _linear_body,
        grid=(10,),
        in_specs=[
            pl.BlockSpec((N // 10, D), lambda i: (i, 0)),
            pl.BlockSpec((D, D), lambda i: (0, 0)),
            pl.BlockSpec((1, D), lambda i: (0, 0)),
        ],
        out_specs=pl.BlockSpec((N // 10, D), lambda i: (i, 0)),
        out_shape=jax.ShapeDtypeStruct((N, D), jnp.float32),
    )(emb, wt, b2)


def kernel(x, edge_index, edge_weight, W_weight, W_bias):
    xp = jnp.concatenate([x, jnp.zeros((NP - N, D), jnp.float32)])
    xs = xp.reshape(NP, NC, HALF).transpose(1, 0, 2).reshape(NC * NP, HALF)
    row = edge_index[0]
    col = edge_index[1]
    pad = E_PAD - E
    colp = jnp.concatenate([col, jnp.zeros((pad,), jnp.int32)])
    rowp = jnp.concatenate([row, jnp.zeros((pad,), jnp.int32)])
    ewp = jnp.concatenate([edge_weight, jnp.zeros((pad,), jnp.float32)])
    colp = colp.reshape(NS, NBLK, BCH, CHUNK)
    rowp = rowp.reshape(NS, NBLK, BCH, CHUNK)
    ewp = ewp.reshape(NS, NBLK, BCH, CHUNK)

    emb_parts = _spmm_call(xs, colp, rowp, ewp)
    emb = emb_parts.transpose(1, 0, 2).reshape(NP, D)[:N]
    return _linear(emb, W_weight.T, W_bias.reshape(1, D))


# speedup vs baseline: 2.4088x; 2.4088x over previous
import functools

import jax
import jax.numpy as jnp
from jax import lax
from jax.experimental import pallas as pl
from jax.experimental.pallas import tpu as pltpu
from jax.experimental.pallas import tpu_sc as plsc

N = 10000
NP = 10240
E = 320000
D = 128
HALF = 64
DEGREE = 16
ALPHA = 0.6

NC = 2
NS = 16
NR = NP // NS
RQ = 5
RB = NR // RQ
CHUNK = 128
BCH = 8
NBLK = 20
NCHUNK = NBLK * BCH
E_TILE = NCHUNK * CHUNK
E_PAD = NS * E_TILE
NRING = 4
LOOK = 2

_GDN = lax.GatherDimensionNumbers(
    offset_dims=(), collapsed_slice_dims=(0,), start_index_map=(0,))


def _lane_bcast(vec, l):
    idx = jnp.full((16, 1), l, jnp.int32)
    return lax.gather(vec, idx, _GDN, (1,),
                      mode=lax.GatherScatterMode.PROMISE_IN_BOUNDS)


def _spmm_body(xs, colh, rowh, ewh, out,
               S1, S2, col_s, row_s, ew_s,
               g0, g1, g2, g3, zbuf,
               sg0, sg1, sg2, sg3, ss0, ss1, ss2, ss3, se0, se1):
    c = lax.axis_index("c")
    s = lax.axis_index("s")
    base = s * NR
    coff = c * NP
    gbufs = (g0, g1, g2, g3)
    sgs = (sg0, sg1, sg2, sg3)
    sss = (ss0, ss1, ss2, ss3)
    ses = (se0, se1)

    def zb(i, carry):
        for q2 in range(4):
            zbuf[i, pl.ds(q2 * 16, 16)] = jnp.zeros((16,), jnp.float32)
        return carry

    lax.fori_loop(0, 64, zb, 0)

    for q in range(RQ):
        pltpu.sync_copy(xs.at[pl.ds(coff + base + q * RB, RB)], g0)
        pltpu.sync_copy(g0, S1.at[pl.ds(base + q * RB, RB)])
        pltpu.sync_copy(g0, out.at[c, pl.ds(base + q * RB, RB)])
    for q in range(2 * RQ):
        pltpu.sync_copy(zbuf, S2.at[pl.ds(base + q * 64, 64)])
    plsc.subcore_barrier()

    def issue_stage(bi, half):
        pltpu.async_copy(colh.at[s, bi], col_s.at[half], ses[0])
        pltpu.async_copy(rowh.at[s, bi], row_s.at[half], ses[0])
        pltpu.async_copy(ewh.at[s, bi], ew_s.at[half], ses[1])

    def wait_stage():
        pltpu.make_async_copy(colh.at[0, 0], col_s.at[0], ses[0]).wait()
        pltpu.make_async_copy(rowh.at[0, 0], row_s.at[0], ses[0]).wait()
        pltpu.make_async_copy(ewh.at[0, 0], ew_s.at[0], ses[1]).wait()

    def wait_sg(b):
        pltpu.make_async_copy(xs.at[pl.ds(0, CHUNK)], gbufs[b], sgs[b]).wait()

    def wait_ss(b):
        pltpu.make_async_copy(xs.at[pl.ds(0, CHUNK)], gbufs[b], sss[b]).wait()

    def one_round(src, dst, asc):
        def process(gkp, pb):
            hp = lax.rem(gkp >> 3, 3)
            kp = gkp & 7
            wait_sg(pb)
            g = gbufs[pb]

            def h8(h, c3):
                wv = ew_s[hp, kp, pl.ds((h >> 1) * 16, 16)]
                lb = (h & 1) * 8
                for dl in range(8):
                    w = _lane_bcast(wv, lb + dl)
                    for q2 in range(4):
                        slx = pl.ds(q2 * 16, 16)
                        g[h * 8 + dl, slx] = g[h * 8 + dl, slx] * w
                return c3

            lax.fori_loop(0, CHUNK // 8, h8, 0)
            pltpu.async_copy(g, dst.at[row_s.at[hp, kp]], sss[pb], add=True)

        issue_stage(0, 0)

        def block_body(bi, carry):
            half = lax.rem(bi, 3)
            wait_stage()

            @pl.when(bi + 1 < NBLK)
            def _():
                issue_stage(bi + 1, lax.rem(bi + 1, 3))

            def kk_body(kk, c2):
                for b in range(NRING):
                    gk = bi * BCH + kk * NRING + b

                    @pl.when(gk >= NRING)
                    def _():
                        wait_ss(b)

                    pltpu.async_copy(
                        src.at[col_s.at[half, kk * NRING + b]],
                        gbufs[b], sgs[b])

                    @pl.when(gk >= LOOK)
                    def _():
                        process(gk - LOOK, (b + NRING - LOOK) % NRING)
                return c2

            lax.fori_loop(0, BCH // NRING, kk_body, 0)
            return carry

        lax.fori_loop(0, NBLK, block_body, 0)
        for t in range(LOOK):
            gkp = NCHUNK - LOOK + t
            process(gkp, gkp % NRING)
        for b in range(NRING):
            wait_ss(b)
        plsc.subcore_barrier()

        def rd(q):
            m = q % 2
            pltpu.async_copy(dst.at[pl.ds(base + q * RB, RB)],
                             gbufs[2 * m], sgs[2 * m])
            pltpu.async_copy(out.at[c, pl.ds(base + q * RB, RB)],
                             gbufs[2 * m + 1], sgs[2 * m + 1])

        rd(0)
        rd(1)
        for q in range(RQ):
            m = q % 2
            wait_sg(2 * m)
            wait_sg(2 * m + 1)
            ga = gbufs[2 * m]
            ge = gbufs[2 * m + 1]

            def eb(i, c2):
                for q2 in range(4):
                    slx = pl.ds(q2 * 16, 16)
                    ge[i, slx] = ge[i, slx] + ga[i, slx] * asc
                return c2

            lax.fori_loop(0, RB, eb, 0)
            pltpu.async_copy(ge, out.at[c, pl.ds(base + q * RB, RB)],
                             sss[2 * m + 1])
            pltpu.async_copy(zbuf, src.at[pl.ds(base + q * RB, 64)], ses[0])
            pltpu.async_copy(zbuf, src.at[pl.ds(base + q * RB + 64, 64)],
                             ses[0])
            if q + 2 <= RQ - 1:
                wait_ss(2 * m + 1)
                rd(q + 2)
        wait_ss(1)
        wait_ss(3)
        for _ in range(2 * RQ):
            pltpu.make_async_copy(zbuf, src.at[pl.ds(base, 64)],
                                  ses[0]).wait()
        plsc.subcore_barrier()

    a = jnp.float32(ALPHA)

    def two_rounds(r2, asc):
        one_round(S1, S2, asc)
        one_round(S2, S1, asc * a)
        return asc * jnp.float32(ALPHA * ALPHA)

    lax.fori_loop(0, DEGREE // 2, two_rounds, a)


_spmm_call = functools.partial(
    pl.kernel,
    out_type=jax.ShapeDtypeStruct((NC, NP, HALF), jnp.float32),
    mesh=plsc.VectorSubcoreMesh(core_axis_name="c", subcore_axis_name="s"),
    compiler_params=pltpu.CompilerParams(use_tc_tiling_on_sc=False),
    scratch_types=(
        [
            pltpu.VMEM_SHARED((NP, HALF), jnp.float32),
            pltpu.VMEM_SHARED((NP, HALF), jnp.float32),
            pltpu.VMEM((3, BCH, CHUNK), jnp.int32),
            pltpu.VMEM((3, BCH, CHUNK), jnp.int32),
            pltpu.VMEM((3, BCH, CHUNK), jnp.float32),
        ]
        + [pltpu.VMEM((CHUNK, HALF), jnp.float32)] * 4
        + [pltpu.VMEM((64, HALF), jnp.float32)]
        + [pltpu.SemaphoreType.DMA] * 10
    ),
)(_spmm_body)


def _linear_body(emb_ref, w_ref, b_ref, o_ref):
    o_ref[...] = (
        jnp.dot(emb_ref[...] * (1.0 / DEGREE), w_ref[...],
                preferred_element_type=jnp.float32)
        + b_ref[...]
    )


def _linear(emb, wt, b2):
    return pl.pallas_call(
        _linear_body,
        grid=(10,),
        in_specs=[
            pl.BlockSpec((N // 10, D), lambda i: (i, 0)),
            pl.BlockSpec((D, D), lambda i: (0, 0)),
            pl.BlockSpec((1, D), lambda i: (0, 0)),
        ],
        out_specs=pl.BlockSpec((N // 10, D), lambda i: (i, 0)),
        out_shape=jax.ShapeDtypeStruct((N, D), jnp.float32),
    )(emb, wt, b2)


def kernel(x, edge_index, edge_weight, W_weight, W_bias):
    xp = jnp.concatenate([x, jnp.zeros((NP - N, D), jnp.float32)])
    xs = xp.reshape(NP, NC, HALF).transpose(1, 0, 2).reshape(NC * NP, HALF)
    row = edge_index[0]
    col = edge_index[1]
    pad = E_PAD - E
    colp = jnp.concatenate([col, jnp.zeros((pad,), jnp.int32)])
    rowp = jnp.concatenate([row, jnp.zeros((pad,), jnp.int32)])
    ewp = jnp.concatenate([edge_weight, jnp.zeros((pad,), jnp.float32)])
    colp = colp.reshape(NS, NBLK, BCH, CHUNK)
    rowp = rowp.reshape(NS, NBLK, BCH, CHUNK)
    ewp = ewp.reshape(NS, NBLK, BCH, CHUNK)

    emb_parts = _spmm_call(xs, colp, rowp, ewp)
    emb = emb_parts.transpose(1, 0, 2).reshape(NP, D)[:N]
    return _linear(emb, W_weight.T, W_bias.reshape(1, D))
